# baseline (speedup 1.0000x reference)
import functools

import jax
import jax.numpy as jnp
import numpy as np
from jax import lax
from jax.experimental import pallas as pl
from jax.experimental.pallas import tpu as pltpu
from jax.experimental.pallas import tpu_sc as plsc

T = 2048
D = 1024
H = 4096
O = 1024
E = 8
K = 2
A = T * K
HB = 512
NH = H // HB
RC = 128

NW = 32
SLOTS_W = A // NW
CHUNK = 64
NCHUNK = SLOTS_W // CHUNK


def _routing_kernel(x_ref, mu_ref, ls_ref, lp_ref, w_ref, ti_ref, wn_ref,
                    dest_ref, offs_ref):
    x = x_ref[...]
    ls = ls_ref[...]
    inv_sigma = jnp.exp(-ls)
    sls = jnp.sum(ls, axis=1)

    cols = []
    for e in range(E):
        d = (x - mu_ref[e, :][None, :]) * inv_sigma[e, :][None, :]
        s = jnp.sum(d * d, axis=1, keepdims=True)
        cols.append(-0.5 * s - sls[e])
    lp = jnp.concatenate(cols, axis=1)
    lp_ref[...] = lp

    m = jnp.max(lp, axis=1, keepdims=True)
    ex = jnp.exp(lp - m)
    w = ex / jnp.sum(ex, axis=1, keepdims=True)
    w_ref[...] = w

    lane = lax.broadcasted_iota(jnp.int32, (T, E), 1)
    m1 = jnp.max(w, axis=1, keepdims=True)
    i1 = jnp.min(jnp.where(w == m1, lane, E), axis=1, keepdims=True)
    h1 = lane == i1
    wm = jnp.where(h1, -1.0, w)
    m2 = jnp.max(wm, axis=1, keepdims=True)
    i2 = jnp.min(jnp.where(wm == m2, lane, E), axis=1, keepdims=True)
    h2 = lane == i2
    ti_ref[...] = jnp.concatenate([i1, i2], axis=1)

    norm = m1 + m2 + 1e-9
    wn1 = (m1 / norm) * jnp.ones((T, 16), jnp.float32)
    wn2 = (m2 / norm) * jnp.ones((T, 16), jnp.float32)
    wn_ref[...] = jnp.concatenate([wn1, wn2], axis=1)

    hh = (h1.astype(jnp.float32) + h2.astype(jnp.float32))
    ri = lax.broadcasted_iota(jnp.int32, (T, T), 0)
    ci = lax.broadcasted_iota(jnp.int32, (T, T), 1)
    tri = (ri > ci).astype(jnp.bfloat16)
    cnt = jnp.dot(tri, hh.astype(jnp.bfloat16),
                  preferred_element_type=jnp.float32)
    totals = jnp.sum(hh, axis=0, keepdims=True)
    e8r = lax.broadcasted_iota(jnp.int32, (E, E), 0)
    e8c = lax.broadcasted_iota(jnp.int32, (E, E), 1)
    offs = jnp.sum(jnp.where(e8c < e8r, jnp.broadcast_to(totals, (E, E)), 0.0),
                   axis=1, keepdims=True).reshape(1, E)
    base = offs + cnt
    d1 = jnp.sum(jnp.where(h1, base, 0.0), axis=1, keepdims=True)
    d2 = jnp.sum(jnp.where(h2, base, 0.0), axis=1, keepdims=True)
    dest_ref[...] = jnp.concatenate([d1, d2], axis=1).astype(jnp.int32)

    lane16 = lax.broadcasted_iota(jnp.int32, (1, 16), 1)
    offs_i = jnp.concatenate(
        [offs.astype(jnp.int32), jnp.full((1, 8), A, jnp.int32)], axis=1)
    offs_ref[...] = jnp.where(lane16 < E, offs_i, A)


def _run_routing(x2, mus, lsig, interpret=False):
    f32 = jnp.float32
    return pl.pallas_call(
        _routing_kernel,
        out_shape=(
            jax.ShapeDtypeStruct((T, E), f32),
            jax.ShapeDtypeStruct((T, E), f32),
            jax.ShapeDtypeStruct((T, K), jnp.int32),
            jax.ShapeDtypeStruct((T, 32), f32),
            jax.ShapeDtypeStruct((T, K), jnp.int32),
            jax.ShapeDtypeStruct((1, 16), jnp.int32),
        ),
        interpret=interpret,
    )(x2, mus, lsig)


def _sc_dispatch(x2, tok3, dest3):
    mesh = plsc.VectorSubcoreMesh(core_axis_name="c", subcore_axis_name="s")

    @functools.partial(
        pl.kernel,
        out_type=jax.ShapeDtypeStruct((A, D), jnp.float32),
        mesh=mesh,
        scratch_types=[
            pltpu.VMEM((CHUNK,), jnp.int32),
            pltpu.VMEM((CHUNK,), jnp.int32),
            pltpu.VMEM((CHUNK, D), jnp.float32),
            pltpu.SemaphoreType.DMA,
        ],
    )
    def k(x_hbm, tok_hbm, dest_hbm, xd_hbm, tok_v, didx_v, rows_v, sem):
        wid = lax.axis_index("c") * 16 + lax.axis_index("s")
        for ch in range(NCHUNK):
            pltpu.sync_copy(tok_hbm.at[wid, ch], tok_v)
            pltpu.sync_copy(dest_hbm.at[wid, ch], didx_v)
            pltpu.async_copy(x_hbm.at[tok_v], rows_v, sem).wait()
            pltpu.async_copy(rows_v, xd_hbm.at[didx_v], sem).wait()

    return k(x2, tok3, dest3)


def _ffn_kernel(offs_ref, xd_ref, w1_ref, b1_ref, w2_ref, b2_ref, out_ref):
    h = pl.program_id(1)
    e = pl.program_id(0)
    start = offs_ref[e]
    end = offs_ref[e + 1]
    c_lo = start // RC
    c_hi = (end + RC - 1) // RC

    w1 = w1_ref[0].astype(jnp.bfloat16)
    w2 = w2_ref[0].astype(jnp.bfloat16)
    b1 = b1_ref[0]
    b2 = b2_ref[0]

    def body(c, carry):
        r0 = pl.multiple_of(c * RC, RC)
        xa = xd_ref[pl.ds(r0, RC), :].astype(jnp.bfloat16)
        hid = jnp.dot(xa, w1, preferred_element_type=jnp.float32) + b1
        hid = hid * 0.5 * (1.0 + lax.erf(hid * np.float32(0.7071067811865476)))
        y = jnp.dot(hid.astype(jnp.bfloat16), w2,
                    preferred_element_type=jnp.float32)
        rowid = r0 + lax.broadcasted_iota(jnp.int32, (RC, 1), 0)
        mask = (rowid >= start) & (rowid < end)

        @pl.when(h == 0)
        def _():
            out_ref[pl.ds(r0, RC), :] = jnp.where(
                mask, y + b2, out_ref[pl.ds(r0, RC), :])

        @pl.when(h != 0)
        def _():
            prev = out_ref[pl.ds(r0, RC), :]
            out_ref[pl.ds(r0, RC), :] = jnp.where(mask, prev + y, prev)

        return carry

    lax.fori_loop(c_lo, c_hi, body, 0)


def _run_ffn(offs, x_disp, W1, b1, W2, b2, interpret=False):
    grid_spec = pltpu.PrefetchScalarGridSpec(
        num_scalar_prefetch=1,
        grid=(E, NH),
        in_specs=[
            pl.BlockSpec((A, D), lambda e, h, offs: (0, 0)),
            pl.BlockSpec((1, D, HB), lambda e, h, offs: (e, 0, h)),
            pl.BlockSpec((1, 1, HB), lambda e, h, offs: (e, 0, h)),
            pl.BlockSpec((1, HB, O), lambda e, h, offs: (e, h, 0)),
            pl.BlockSpec((1, 1, O), lambda e, h, offs: (e, 0, 0)),
        ],
        out_specs=pl.BlockSpec((A, O), lambda e, h, offs: (0, 0)),
    )
    return pl.pallas_call(
        _ffn_kernel,
        grid_spec=grid_spec,
        out_shape=jax.ShapeDtypeStruct((A, O), jnp.float32),
        interpret=interpret,
    )(offs, x_disp, W1, b1.reshape(E, 1, H), W2, b2.reshape(E, 1, O))


def _sc_combine(y_disp, dest3, wnb):
    mesh = plsc.VectorSubcoreMesh(core_axis_name="c", subcore_axis_name="s")
    TOK_CH = CHUNK // K

    @functools.partial(
        pl.kernel,
        out_type=jax.ShapeDtypeStruct((T, O), jnp.float32),
        mesh=mesh,
        scratch_types=[
            pltpu.VMEM((CHUNK,), jnp.int32),
            pltpu.VMEM((CHUNK, O), jnp.float32),
            pltpu.VMEM((CHUNK, 16), jnp.float32),
            pltpu.VMEM((TOK_CH, O), jnp.float32),
            pltpu.SemaphoreType.DMA,
        ],
    )
    def k(y_hbm, dest_hbm, wnb_hbm, out_hbm, didx_v, g_v, wv, o_v, sem):
        wid = lax.axis_index("c") * 16 + lax.axis_index("s")
        for ch in range(NCHUNK):
            base_slot = pl.multiple_of(wid * SLOTS_W + ch * CHUNK, CHUNK)
            base_tok = pl.multiple_of(wid * SLOTS_W // K + ch * CHUNK // K,
                                      CHUNK // K)
            pltpu.sync_copy(dest_hbm.at[wid, ch], didx_v)
            pltpu.async_copy(y_hbm.at[didx_v], g_v, sem).wait()
            pltpu.sync_copy(wnb_hbm.at[pl.ds(base_slot, CHUNK)], wv)

            @pl.loop(0, TOK_CH)
            def _(i):
                w0 = wv[2 * i, :]
                w1 = wv[2 * i + 1, :]

                @pl.loop(0, O, step=16)
                def _(c):
                    o_v[i, pl.ds(c, 16)] = (
                        w0 * g_v[2 * i, pl.ds(c, 16)]
                        + w1 * g_v[2 * i + 1, pl.ds(c, 16)])

            pltpu.sync_copy(o_v, out_hbm.at[pl.ds(base_tok, TOK_CH)])

    return k(y_disp, dest3, wnb)


_TOK3 = np.arange(A, dtype=np.int32).reshape(NW, NCHUNK, CHUNK) // K


def kernel(x, expert_mus, expert_log_sigmas, W1, b1, W2, b2):
    b, s, d = x.shape
    x2 = x.reshape(T, D)

    lp, w, ti, wnb, dest, offs16 = _run_routing(
        x2, expert_mus, expert_log_sigmas)

    dest3 = dest.reshape(NW, NCHUNK, CHUNK)
    tok3 = jnp.asarray(_TOK3)

    x_disp = _sc_dispatch(x2, tok3, dest3)

    offs = offs16[0, :E + 1]
    y_disp = _run_ffn(offs, x_disp, W1, b1, W2, b2)

    wnb_flat = wnb.reshape(A, 16)
    final = _sc_combine(y_disp, dest3, wnb_flat)

    return (final.reshape(b, s, O),
            lp.reshape(b, s, E),
            w.reshape(b, s, E),
            ti.reshape(b, s, K))


# speedup vs baseline: 1.1226x; 1.1226x over previous
import functools

import jax
import jax.numpy as jnp
import numpy as np
from jax import lax
from jax.experimental import pallas as pl
from jax.experimental.pallas import tpu as pltpu
from jax.experimental.pallas import tpu_sc as plsc

T = 2048
D = 1024
H = 4096
O = 1024
E = 8
K = 2
A = T * K
HB = 512
NH = H // HB
RC = 256

NW = 32
SLOTS_W = A // NW
CHUNK = 64
NCHUNK = SLOTS_W // CHUNK


def _routing_kernel(x_ref, mu_ref, ls_ref, lp_ref, w_ref, ti_ref, wn_ref,
                    dest_ref, offs_ref):
    x = x_ref[...]
    ls = ls_ref[...]
    inv_sigma = jnp.exp(-ls)
    sls = jnp.sum(ls, axis=1)

    cols = []
    for e in range(E):
        d = (x - mu_ref[e, :][None, :]) * inv_sigma[e, :][None, :]
        s = jnp.sum(d * d, axis=1, keepdims=True)
        cols.append(-0.5 * s - sls[e])
    lp = jnp.concatenate(cols, axis=1)
    lp_ref[...] = lp

    m = jnp.max(lp, axis=1, keepdims=True)
    ex = jnp.exp(lp - m)
    w = ex / jnp.sum(ex, axis=1, keepdims=True)
    w_ref[...] = w

    lane = lax.broadcasted_iota(jnp.int32, (T, E), 1)
    m1 = jnp.max(w, axis=1, keepdims=True)
    i1 = jnp.min(jnp.where(w == m1, lane, E), axis=1, keepdims=True)
    h1 = lane == i1
    wm = jnp.where(h1, -1.0, w)
    m2 = jnp.max(wm, axis=1, keepdims=True)
    i2 = jnp.min(jnp.where(wm == m2, lane, E), axis=1, keepdims=True)
    h2 = lane == i2
    ti_ref[...] = jnp.concatenate([i1, i2], axis=1)

    norm = m1 + m2 + 1e-9
    wn1 = (m1 / norm) * jnp.ones((T, 16), jnp.float32)
    wn2 = (m2 / norm) * jnp.ones((T, 16), jnp.float32)
    wn_ref[...] = jnp.concatenate([wn1, wn2], axis=1)

    hh = (h1.astype(jnp.float32) + h2.astype(jnp.float32))
    ri = lax.broadcasted_iota(jnp.int32, (T, T), 0)
    ci = lax.broadcasted_iota(jnp.int32, (T, T), 1)
    tri = (ri > ci).astype(jnp.bfloat16)
    cnt = jnp.dot(tri, hh.astype(jnp.bfloat16),
                  preferred_element_type=jnp.float32)
    totals = jnp.sum(hh, axis=0, keepdims=True)
    e8r = lax.broadcasted_iota(jnp.int32, (E, E), 0)
    e8c = lax.broadcasted_iota(jnp.int32, (E, E), 1)
    offs = jnp.sum(jnp.where(e8c < e8r, jnp.broadcast_to(totals, (E, E)), 0.0),
                   axis=1, keepdims=True).reshape(1, E)
    base = offs + cnt
    d1 = jnp.sum(jnp.where(h1, base, 0.0), axis=1, keepdims=True)
    d2 = jnp.sum(jnp.where(h2, base, 0.0), axis=1, keepdims=True)
    dest_ref[...] = jnp.concatenate([d1, d2], axis=1).astype(jnp.int32)

    lane16 = lax.broadcasted_iota(jnp.int32, (1, 16), 1)
    offs_i = jnp.concatenate(
        [offs.astype(jnp.int32), jnp.full((1, 8), A, jnp.int32)], axis=1)
    offs_ref[...] = jnp.where(lane16 < E, offs_i, A)


def _run_routing(x2, mus, lsig, interpret=False):
    f32 = jnp.float32
    return pl.pallas_call(
        _routing_kernel,
        out_shape=(
            jax.ShapeDtypeStruct((T, E), f32),
            jax.ShapeDtypeStruct((T, E), f32),
            jax.ShapeDtypeStruct((T, K), jnp.int32),
            jax.ShapeDtypeStruct((T, 32), f32),
            jax.ShapeDtypeStruct((T, K), jnp.int32),
            jax.ShapeDtypeStruct((1, 16), jnp.int32),
        ),
        interpret=interpret,
    )(x2, mus, lsig)


def _sc_dispatch(x2, tok3, dest3):
    mesh = plsc.VectorSubcoreMesh(core_axis_name="c", subcore_axis_name="s")

    @functools.partial(
        pl.kernel,
        out_type=jax.ShapeDtypeStruct((A, D), jnp.float32),
        mesh=mesh,
        scratch_types=[
            pltpu.VMEM((CHUNK,), jnp.int32),
            pltpu.VMEM((CHUNK,), jnp.int32),
            pltpu.VMEM((CHUNK, D), jnp.float32),
            pltpu.SemaphoreType.DMA,
        ],
    )
    def k(x_hbm, tok_hbm, dest_hbm, xd_hbm, tok_v, didx_v, rows_v, sem):
        wid = lax.axis_index("c") * 16 + lax.axis_index("s")
        for ch in range(NCHUNK):
            pltpu.sync_copy(tok_hbm.at[wid, ch], tok_v)
            pltpu.sync_copy(dest_hbm.at[wid, ch], didx_v)
            pltpu.async_copy(x_hbm.at[tok_v], rows_v, sem).wait()
            pltpu.async_copy(rows_v, xd_hbm.at[didx_v], sem).wait()

    return k(x2, tok3, dest3)


def _ffn_kernel(offs_ref, xd_ref, w1_ref, b1_ref, w2_ref, b2_ref, out_ref):
    h = pl.program_id(1)
    e = pl.program_id(0)
    start = offs_ref[e]
    end = offs_ref[e + 1]
    c_lo = start // RC
    c_hi = (end + RC - 1) // RC

    w1 = w1_ref[0].astype(jnp.bfloat16)
    w2 = w2_ref[0].astype(jnp.bfloat16)
    b1 = b1_ref[0]
    b2 = b2_ref[0]

    def body(c, carry):
        r0 = pl.multiple_of(c * RC, RC)
        xa = xd_ref[pl.ds(r0, RC), :].astype(jnp.bfloat16)
        hid = jnp.dot(xa, w1, preferred_element_type=jnp.float32) + b1
        hid = hid * 0.5 * (1.0 + lax.erf(hid * np.float32(0.7071067811865476)))
        y = jnp.dot(hid.astype(jnp.bfloat16), w2,
                    preferred_element_type=jnp.float32)
        rowid = r0 + lax.broadcasted_iota(jnp.int32, (RC, 1), 0)
        mask = (rowid >= start) & (rowid < end)

        @pl.when(h == 0)
        def _():
            out_ref[pl.ds(r0, RC), :] = jnp.where(
                mask, y + b2, out_ref[pl.ds(r0, RC), :])

        @pl.when(h != 0)
        def _():
            prev = out_ref[pl.ds(r0, RC), :]
            out_ref[pl.ds(r0, RC), :] = jnp.where(mask, prev + y, prev)

        return carry

    lax.fori_loop(c_lo, c_hi, body, 0)


def _run_ffn(offs, x_disp, W1, b1, W2, b2, interpret=False):
    grid_spec = pltpu.PrefetchScalarGridSpec(
        num_scalar_prefetch=1,
        grid=(E, NH),
        in_specs=[
            pl.BlockSpec((A, D), lambda e, h, offs: (0, 0)),
            pl.BlockSpec((1, D, HB), lambda e, h, offs: (e, 0, h)),
            pl.BlockSpec((1, 1, HB), lambda e, h, offs: (e, 0, h)),
            pl.BlockSpec((1, HB, O), lambda e, h, offs: (e, h, 0)),
            pl.BlockSpec((1, 1, O), lambda e, h, offs: (e, 0, 0)),
        ],
        out_specs=pl.BlockSpec((A, O), lambda e, h, offs: (0, 0)),
    )
    return pl.pallas_call(
        _ffn_kernel,
        grid_spec=grid_spec,
        out_shape=jax.ShapeDtypeStruct((A, O), jnp.float32),
        interpret=interpret,
    )(offs, x_disp, W1, b1.reshape(E, 1, H), W2, b2.reshape(E, 1, O))


def _sc_combine(y_disp, dest3, wnb):
    mesh = plsc.VectorSubcoreMesh(core_axis_name="c", subcore_axis_name="s")
    TOK_CH = CHUNK // K

    @functools.partial(
        pl.kernel,
        out_type=jax.ShapeDtypeStruct((T, O), jnp.float32),
        mesh=mesh,
        scratch_types=[
            pltpu.VMEM((CHUNK,), jnp.int32),
            pltpu.VMEM((CHUNK, O), jnp.float32),
            pltpu.VMEM((CHUNK, 16), jnp.float32),
            pltpu.VMEM((TOK_CH, O), jnp.float32),
            pltpu.SemaphoreType.DMA,
        ],
    )
    def k(y_hbm, dest_hbm, wnb_hbm, out_hbm, didx_v, g_v, wv, o_v, sem):
        wid = lax.axis_index("c") * 16 + lax.axis_index("s")
        for ch in range(NCHUNK):
            base_slot = pl.multiple_of(wid * SLOTS_W + ch * CHUNK, CHUNK)
            base_tok = pl.multiple_of(wid * SLOTS_W // K + ch * CHUNK // K,
                                      CHUNK // K)
            pltpu.sync_copy(dest_hbm.at[wid, ch], didx_v)
            pltpu.async_copy(y_hbm.at[didx_v], g_v, sem).wait()
            pltpu.sync_copy(wnb_hbm.at[pl.ds(base_slot, CHUNK)], wv)

            @pl.loop(0, TOK_CH)
            def _(i):
                w0 = wv[2 * i, :]
                w1 = wv[2 * i + 1, :]

                @pl.loop(0, O, step=16)
                def _(c):
                    o_v[i, pl.ds(c, 16)] = (
                        w0 * g_v[2 * i, pl.ds(c, 16)]
                        + w1 * g_v[2 * i + 1, pl.ds(c, 16)])

            pltpu.sync_copy(o_v, out_hbm.at[pl.ds(base_tok, TOK_CH)])

    return k(y_disp, dest3, wnb)


_TOK3 = np.arange(A, dtype=np.int32).reshape(NW, NCHUNK, CHUNK) // K


def kernel(x, expert_mus, expert_log_sigmas, W1, b1, W2, b2):
    b, s, d = x.shape
    x2 = x.reshape(T, D)

    lp, w, ti, wnb, dest, offs16 = _run_routing(
        x2, expert_mus, expert_log_sigmas)

    dest3 = dest.reshape(NW, NCHUNK, CHUNK)
    tok3 = jnp.asarray(_TOK3)

    x_disp = _sc_dispatch(x2, tok3, dest3)

    offs = offs16[0, :E + 1]
    y_disp = _run_ffn(offs, x_disp, W1, b1, W2, b2)

    wnb_flat = wnb.reshape(A, 16)
    final = _sc_combine(y_disp, dest3, wnb_flat)

    return (final.reshape(b, s, O),
            lp.reshape(b, s, E),
            w.reshape(b, s, E),
            ti.reshape(b, s, K))


# speedup vs baseline: 1.1248x; 1.0020x over previous
import functools

import jax
import jax.numpy as jnp
import numpy as np
from jax import lax
from jax.experimental import pallas as pl
from jax.experimental.pallas import tpu as pltpu
from jax.experimental.pallas import tpu_sc as plsc

T = 2048
D = 1024
H = 4096
O = 1024
E = 8
K = 2
A = T * K
HB = 512
NH = H // HB
RC = 512

NW = 32
SLOTS_W = A // NW
CHUNK = 64
NCHUNK = SLOTS_W // CHUNK


def _routing_kernel(x_ref, mu_ref, ls_ref, lp_ref, w_ref, ti_ref, wn_ref,
                    dest_ref, offs_ref):
    x = x_ref[...]
    ls = ls_ref[...]
    inv_sigma = jnp.exp(-ls)
    sls = jnp.sum(ls, axis=1)

    cols = []
    for e in range(E):
        d = (x - mu_ref[e, :][None, :]) * inv_sigma[e, :][None, :]
        s = jnp.sum(d * d, axis=1, keepdims=True)
        cols.append(-0.5 * s - sls[e])
    lp = jnp.concatenate(cols, axis=1)
    lp_ref[...] = lp

    m = jnp.max(lp, axis=1, keepdims=True)
    ex = jnp.exp(lp - m)
    w = ex / jnp.sum(ex, axis=1, keepdims=True)
    w_ref[...] = w

    lane = lax.broadcasted_iota(jnp.int32, (T, E), 1)
    m1 = jnp.max(w, axis=1, keepdims=True)
    i1 = jnp.min(jnp.where(w == m1, lane, E), axis=1, keepdims=True)
    h1 = lane == i1
    wm = jnp.where(h1, -1.0, w)
    m2 = jnp.max(wm, axis=1, keepdims=True)
    i2 = jnp.min(jnp.where(wm == m2, lane, E), axis=1, keepdims=True)
    h2 = lane == i2
    ti_ref[...] = jnp.concatenate([i1, i2], axis=1)

    norm = m1 + m2 + 1e-9
    wn1 = (m1 / norm) * jnp.ones((T, 16), jnp.float32)
    wn2 = (m2 / norm) * jnp.ones((T, 16), jnp.float32)
    wn_ref[...] = jnp.concatenate([wn1, wn2], axis=1)

    hh = (h1.astype(jnp.float32) + h2.astype(jnp.float32))
    ri = lax.broadcasted_iota(jnp.int32, (T, T), 0)
    ci = lax.broadcasted_iota(jnp.int32, (T, T), 1)
    tri = (ri > ci).astype(jnp.bfloat16)
    cnt = jnp.dot(tri, hh.astype(jnp.bfloat16),
                  preferred_element_type=jnp.float32)
    totals = jnp.sum(hh, axis=0, keepdims=True)
    e8r = lax.broadcasted_iota(jnp.int32, (E, E), 0)
    e8c = lax.broadcasted_iota(jnp.int32, (E, E), 1)
    offs = jnp.sum(jnp.where(e8c < e8r, jnp.broadcast_to(totals, (E, E)), 0.0),
                   axis=1, keepdims=True).reshape(1, E)
    base = offs + cnt
    d1 = jnp.sum(jnp.where(h1, base, 0.0), axis=1, keepdims=True)
    d2 = jnp.sum(jnp.where(h2, base, 0.0), axis=1, keepdims=True)
    dest_ref[...] = jnp.concatenate([d1, d2], axis=1).astype(jnp.int32)

    lane16 = lax.broadcasted_iota(jnp.int32, (1, 16), 1)
    offs_i = jnp.concatenate(
        [offs.astype(jnp.int32), jnp.full((1, 8), A, jnp.int32)], axis=1)
    offs_ref[...] = jnp.where(lane16 < E, offs_i, A)


def _run_routing(x2, mus, lsig, interpret=False):
    f32 = jnp.float32
    return pl.pallas_call(
        _routing_kernel,
        out_shape=(
            jax.ShapeDtypeStruct((T, E), f32),
            jax.ShapeDtypeStruct((T, E), f32),
            jax.ShapeDtypeStruct((T, K), jnp.int32),
            jax.ShapeDtypeStruct((T, 32), f32),
            jax.ShapeDtypeStruct((T, K), jnp.int32),
            jax.ShapeDtypeStruct((1, 16), jnp.int32),
        ),
        interpret=interpret,
    )(x2, mus, lsig)


def _sc_dispatch(x2, tok3, dest3):
    mesh = plsc.VectorSubcoreMesh(core_axis_name="c", subcore_axis_name="s")

    @functools.partial(
        pl.kernel,
        out_type=jax.ShapeDtypeStruct((A, D), jnp.float32),
        mesh=mesh,
        scratch_types=[
            pltpu.VMEM((CHUNK,), jnp.int32),
            pltpu.VMEM((CHUNK,), jnp.int32),
            pltpu.VMEM((CHUNK, D), jnp.float32),
            pltpu.SemaphoreType.DMA,
        ],
    )
    def k(x_hbm, tok_hbm, dest_hbm, xd_hbm, tok_v, didx_v, rows_v, sem):
        wid = lax.axis_index("c") * 16 + lax.axis_index("s")
        for ch in range(NCHUNK):
            pltpu.sync_copy(tok_hbm.at[wid, ch], tok_v)
            pltpu.sync_copy(dest_hbm.at[wid, ch], didx_v)
            pltpu.async_copy(x_hbm.at[tok_v], rows_v, sem).wait()
            pltpu.async_copy(rows_v, xd_hbm.at[didx_v], sem).wait()

    return k(x2, tok3, dest3)


def _ffn_kernel(offs_ref, xd_ref, w1_ref, b1_ref, w2_ref, b2_ref, out_ref):
    h = pl.program_id(1)
    e = pl.program_id(0)
    start = offs_ref[e]
    end = offs_ref[e + 1]
    c_lo = start // RC
    c_hi = (end + RC - 1) // RC

    w1 = w1_ref[0].astype(jnp.bfloat16)
    w2 = w2_ref[0].astype(jnp.bfloat16)
    b1 = b1_ref[0]
    b2 = b2_ref[0]

    def body(c, carry):
        r0 = pl.multiple_of(c * RC, RC)
        xa = xd_ref[pl.ds(r0, RC), :].astype(jnp.bfloat16)
        hid = jnp.dot(xa, w1, preferred_element_type=jnp.float32) + b1
        hid = hid * 0.5 * (1.0 + lax.erf(hid * np.float32(0.7071067811865476)))
        y = jnp.dot(hid.astype(jnp.bfloat16), w2,
                    preferred_element_type=jnp.float32)
        rowid = r0 + lax.broadcasted_iota(jnp.int32, (RC, 1), 0)
        mask = (rowid >= start) & (rowid < end)

        @pl.when(h == 0)
        def _():
            out_ref[pl.ds(r0, RC), :] = jnp.where(
                mask, y + b2, out_ref[pl.ds(r0, RC), :])

        @pl.when(h != 0)
        def _():
            prev = out_ref[pl.ds(r0, RC), :]
            out_ref[pl.ds(r0, RC), :] = jnp.where(mask, prev + y, prev)

        return carry

    lax.fori_loop(c_lo, c_hi, body, 0)


def _run_ffn(offs, x_disp, W1, b1, W2, b2, interpret=False):
    grid_spec = pltpu.PrefetchScalarGridSpec(
        num_scalar_prefetch=1,
        grid=(E, NH),
        in_specs=[
            pl.BlockSpec((A, D), lambda e, h, offs: (0, 0)),
            pl.BlockSpec((1, D, HB), lambda e, h, offs: (e, 0, h)),
            pl.BlockSpec((1, 1, HB), lambda e, h, offs: (e, 0, h)),
            pl.BlockSpec((1, HB, O), lambda e, h, offs: (e, h, 0)),
            pl.BlockSpec((1, 1, O), lambda e, h, offs: (e, 0, 0)),
        ],
        out_specs=pl.BlockSpec((A, O), lambda e, h, offs: (0, 0)),
    )
    return pl.pallas_call(
        _ffn_kernel,
        grid_spec=grid_spec,
        out_shape=jax.ShapeDtypeStruct((A, O), jnp.float32),
        interpret=interpret,
    )(offs, x_disp, W1, b1.reshape(E, 1, H), W2, b2.reshape(E, 1, O))


def _sc_combine(y_disp, dest3, wnb):
    mesh = plsc.VectorSubcoreMesh(core_axis_name="c", subcore_axis_name="s")
    TOK_CH = CHUNK // K

    @functools.partial(
        pl.kernel,
        out_type=jax.ShapeDtypeStruct((T, O), jnp.float32),
        mesh=mesh,
        scratch_types=[
            pltpu.VMEM((CHUNK,), jnp.int32),
            pltpu.VMEM((CHUNK, O), jnp.float32),
            pltpu.VMEM((CHUNK, 16), jnp.float32),
            pltpu.VMEM((TOK_CH, O), jnp.float32),
            pltpu.SemaphoreType.DMA,
        ],
    )
    def k(y_hbm, dest_hbm, wnb_hbm, out_hbm, didx_v, g_v, wv, o_v, sem):
        wid = lax.axis_index("c") * 16 + lax.axis_index("s")
        for ch in range(NCHUNK):
            base_slot = pl.multiple_of(wid * SLOTS_W + ch * CHUNK, CHUNK)
            base_tok = pl.multiple_of(wid * SLOTS_W // K + ch * CHUNK // K,
                                      CHUNK // K)
            pltpu.sync_copy(dest_hbm.at[wid, ch], didx_v)
            pltpu.async_copy(y_hbm.at[didx_v], g_v, sem).wait()
            pltpu.sync_copy(wnb_hbm.at[pl.ds(base_slot, CHUNK)], wv)

            @pl.loop(0, TOK_CH)
            def _(i):
                w0 = wv[2 * i, :]
                w1 = wv[2 * i + 1, :]

                @pl.loop(0, O, step=16)
                def _(c):
                    o_v[i, pl.ds(c, 16)] = (
                        w0 * g_v[2 * i, pl.ds(c, 16)]
                        + w1 * g_v[2 * i + 1, pl.ds(c, 16)])

            pltpu.sync_copy(o_v, out_hbm.at[pl.ds(base_tok, TOK_CH)])

    return k(y_disp, dest3, wnb)


_TOK3 = np.arange(A, dtype=np.int32).reshape(NW, NCHUNK, CHUNK) // K


def kernel(x, expert_mus, expert_log_sigmas, W1, b1, W2, b2):
    b, s, d = x.shape
    x2 = x.reshape(T, D)

    lp, w, ti, wnb, dest, offs16 = _run_routing(
        x2, expert_mus, expert_log_sigmas)

    dest3 = dest.reshape(NW, NCHUNK, CHUNK)
    tok3 = jnp.asarray(_TOK3)

    x_disp = _sc_dispatch(x2, tok3, dest3)

    offs = offs16[0, :E + 1]
    y_disp = _run_ffn(offs, x_disp, W1, b1, W2, b2)

    wnb_flat = wnb.reshape(A, 16)
    final = _sc_combine(y_disp, dest3, wnb_flat)

    return (final.reshape(b, s, O),
            lp.reshape(b, s, E),
            w.reshape(b, s, E),
            ti.reshape(b, s, K))


# speedup vs baseline: 1.2899x; 1.1468x over previous
import functools

import jax
import jax.numpy as jnp
import numpy as np
from jax import lax
from jax.experimental import pallas as pl
from jax.experimental.pallas import tpu as pltpu
from jax.experimental.pallas import tpu_sc as plsc

T = 2048
D = 1024
H = 4096
O = 1024
E = 8
K = 2
A = T * K
HB = 1024
NH = H // HB
RC = 512

NW = 32
SLOTS_W = A // NW
CHUNK = 64
NCHUNK = SLOTS_W // CHUNK


def _routing_kernel(x_ref, mu_ref, ls_ref, lp_ref, w_ref, ti_ref, wn_ref,
                    dest_ref, offs_ref):
    x = x_ref[...]
    ls = ls_ref[...]
    inv_sigma = jnp.exp(-ls)
    sls = jnp.sum(ls, axis=1)

    cols = []
    for e in range(E):
        d = (x - mu_ref[e, :][None, :]) * inv_sigma[e, :][None, :]
        s = jnp.sum(d * d, axis=1, keepdims=True)
        cols.append(-0.5 * s - sls[e])
    lp = jnp.concatenate(cols, axis=1)
    lp_ref[...] = lp

    m = jnp.max(lp, axis=1, keepdims=True)
    ex = jnp.exp(lp - m)
    w = ex / jnp.sum(ex, axis=1, keepdims=True)
    w_ref[...] = w

    lane = lax.broadcasted_iota(jnp.int32, (T, E), 1)
    m1 = jnp.max(w, axis=1, keepdims=True)
    i1 = jnp.min(jnp.where(w == m1, lane, E), axis=1, keepdims=True)
    h1 = lane == i1
    wm = jnp.where(h1, -1.0, w)
    m2 = jnp.max(wm, axis=1, keepdims=True)
    i2 = jnp.min(jnp.where(wm == m2, lane, E), axis=1, keepdims=True)
    h2 = lane == i2
    ti_ref[...] = jnp.concatenate([i1, i2], axis=1)

    norm = m1 + m2 + 1e-9
    wn1 = (m1 / norm) * jnp.ones((T, 16), jnp.float32)
    wn2 = (m2 / norm) * jnp.ones((T, 16), jnp.float32)
    wn_ref[...] = jnp.concatenate([wn1, wn2], axis=1)

    hh = (h1.astype(jnp.float32) + h2.astype(jnp.float32))
    ri = lax.broadcasted_iota(jnp.int32, (T, T), 0)
    ci = lax.broadcasted_iota(jnp.int32, (T, T), 1)
    tri = (ri > ci).astype(jnp.bfloat16)
    cnt = jnp.dot(tri, hh.astype(jnp.bfloat16),
                  preferred_element_type=jnp.float32)
    totals = jnp.sum(hh, axis=0, keepdims=True)
    e8r = lax.broadcasted_iota(jnp.int32, (E, E), 0)
    e8c = lax.broadcasted_iota(jnp.int32, (E, E), 1)
    offs = jnp.sum(jnp.where(e8c < e8r, jnp.broadcast_to(totals, (E, E)), 0.0),
                   axis=1, keepdims=True).reshape(1, E)
    base = offs + cnt
    d1 = jnp.sum(jnp.where(h1, base, 0.0), axis=1, keepdims=True)
    d2 = jnp.sum(jnp.where(h2, base, 0.0), axis=1, keepdims=True)
    dest_ref[...] = jnp.concatenate([d1, d2], axis=1).astype(jnp.int32)

    lane16 = lax.broadcasted_iota(jnp.int32, (1, 16), 1)
    offs_i = jnp.concatenate(
        [offs.astype(jnp.int32), jnp.full((1, 8), A, jnp.int32)], axis=1)
    offs_ref[...] = jnp.where(lane16 < E, offs_i, A)


def _run_routing(x2, mus, lsig, interpret=False):
    f32 = jnp.float32
    return pl.pallas_call(
        _routing_kernel,
        out_shape=(
            jax.ShapeDtypeStruct((T, E), f32),
            jax.ShapeDtypeStruct((T, E), f32),
            jax.ShapeDtypeStruct((T, K), jnp.int32),
            jax.ShapeDtypeStruct((T, 32), f32),
            jax.ShapeDtypeStruct((T, K), jnp.int32),
            jax.ShapeDtypeStruct((1, 16), jnp.int32),
        ),
        interpret=interpret,
    )(x2, mus, lsig)


def _sc_dispatch(x2, tok3, dest3):
    mesh = plsc.VectorSubcoreMesh(core_axis_name="c", subcore_axis_name="s")

    @functools.partial(
        pl.kernel,
        out_type=jax.ShapeDtypeStruct((A, D), jnp.float32),
        mesh=mesh,
        scratch_types=[
            pltpu.VMEM((CHUNK,), jnp.int32),
            pltpu.VMEM((CHUNK,), jnp.int32),
            pltpu.VMEM((CHUNK, D), jnp.float32),
            pltpu.SemaphoreType.DMA,
        ],
    )
    def k(x_hbm, tok_hbm, dest_hbm, xd_hbm, tok_v, didx_v, rows_v, sem):
        wid = lax.axis_index("c") * 16 + lax.axis_index("s")
        for ch in range(NCHUNK):
            pltpu.sync_copy(tok_hbm.at[wid, ch], tok_v)
            pltpu.sync_copy(dest_hbm.at[wid, ch], didx_v)
            pltpu.async_copy(x_hbm.at[tok_v], rows_v, sem).wait()
            pltpu.async_copy(rows_v, xd_hbm.at[didx_v], sem).wait()

    return k(x2, tok3, dest3)


def _ffn_kernel(offs_ref, xd_ref, w1_ref, b1_ref, w2_ref, b2_ref, out_ref):
    h = pl.program_id(1)
    e = pl.program_id(0)
    start = offs_ref[e]
    end = offs_ref[e + 1]
    c_lo = start // RC
    c_hi = (end + RC - 1) // RC

    w1 = w1_ref[0].astype(jnp.bfloat16)
    w2 = w2_ref[0].astype(jnp.bfloat16)
    b1 = b1_ref[0]
    b2 = b2_ref[0]

    def body(c, carry):
        r0 = pl.multiple_of(c * RC, RC)
        xa = xd_ref[pl.ds(r0, RC), :].astype(jnp.bfloat16)
        hid = jnp.dot(xa, w1, preferred_element_type=jnp.float32) + b1
        hid = hid * 0.5 * (1.0 + lax.erf(hid * np.float32(0.7071067811865476)))
        y = jnp.dot(hid.astype(jnp.bfloat16), w2,
                    preferred_element_type=jnp.float32)
        rowid = r0 + lax.broadcasted_iota(jnp.int32, (RC, 1), 0)
        mask = (rowid >= start) & (rowid < end)

        @pl.when(h == 0)
        def _():
            out_ref[pl.ds(r0, RC), :] = jnp.where(
                mask, y + b2, out_ref[pl.ds(r0, RC), :])

        @pl.when(h != 0)
        def _():
            prev = out_ref[pl.ds(r0, RC), :]
            out_ref[pl.ds(r0, RC), :] = jnp.where(mask, prev + y, prev)

        return carry

    lax.fori_loop(c_lo, c_hi, body, 0)


def _run_ffn(offs, x_disp, W1, b1, W2, b2, interpret=False):
    grid_spec = pltpu.PrefetchScalarGridSpec(
        num_scalar_prefetch=1,
        grid=(E, NH),
        in_specs=[
            pl.BlockSpec((A, D), lambda e, h, offs: (0, 0)),
            pl.BlockSpec((1, D, HB), lambda e, h, offs: (e, 0, h)),
            pl.BlockSpec((1, 1, HB), lambda e, h, offs: (e, 0, h)),
            pl.BlockSpec((1, HB, O), lambda e, h, offs: (e, h, 0)),
            pl.BlockSpec((1, 1, O), lambda e, h, offs: (e, 0, 0)),
        ],
        out_specs=pl.BlockSpec((A, O), lambda e, h, offs: (0, 0)),
    )
    return pl.pallas_call(
        _ffn_kernel,
        grid_spec=grid_spec,
        out_shape=jax.ShapeDtypeStruct((A, O), jnp.float32),
        interpret=interpret,
    )(offs, x_disp, W1, b1.reshape(E, 1, H), W2, b2.reshape(E, 1, O))


def _sc_combine(y_disp, dest3, wnb):
    mesh = plsc.VectorSubcoreMesh(core_axis_name="c", subcore_axis_name="s")
    TOK_CH = CHUNK // K

    @functools.partial(
        pl.kernel,
        out_type=jax.ShapeDtypeStruct((T, O), jnp.float32),
        mesh=mesh,
        scratch_types=[
            pltpu.VMEM((CHUNK,), jnp.int32),
            pltpu.VMEM((CHUNK, O), jnp.float32),
            pltpu.VMEM((CHUNK, 16), jnp.float32),
            pltpu.VMEM((TOK_CH, O), jnp.float32),
            pltpu.SemaphoreType.DMA,
        ],
    )
    def k(y_hbm, dest_hbm, wnb_hbm, out_hbm, didx_v, g_v, wv, o_v, sem):
        wid = lax.axis_index("c") * 16 + lax.axis_index("s")
        for ch in range(NCHUNK):
            base_slot = pl.multiple_of(wid * SLOTS_W + ch * CHUNK, CHUNK)
            base_tok = pl.multiple_of(wid * SLOTS_W // K + ch * CHUNK // K,
                                      CHUNK // K)
            pltpu.sync_copy(dest_hbm.at[wid, ch], didx_v)
            pltpu.async_copy(y_hbm.at[didx_v], g_v, sem).wait()
            pltpu.sync_copy(wnb_hbm.at[pl.ds(base_slot, CHUNK)], wv)

            @pl.loop(0, TOK_CH)
            def _(i):
                w0 = wv[2 * i, :]
                w1 = wv[2 * i + 1, :]

                @pl.loop(0, O, step=16)
                def _(c):
                    o_v[i, pl.ds(c, 16)] = (
                        w0 * g_v[2 * i, pl.ds(c, 16)]
                        + w1 * g_v[2 * i + 1, pl.ds(c, 16)])

            pltpu.sync_copy(o_v, out_hbm.at[pl.ds(base_tok, TOK_CH)])

    return k(y_disp, dest3, wnb)


_TOK3 = np.arange(A, dtype=np.int32).reshape(NW, NCHUNK, CHUNK) // K


def kernel(x, expert_mus, expert_log_sigmas, W1, b1, W2, b2):
    b, s, d = x.shape
    x2 = x.reshape(T, D)

    lp, w, ti, wnb, dest, offs16 = _run_routing(
        x2, expert_mus, expert_log_sigmas)

    dest3 = dest.reshape(NW, NCHUNK, CHUNK)
    tok3 = jnp.asarray(_TOK3)

    x_disp = _sc_dispatch(x2, tok3, dest3)

    offs = offs16[0, :E + 1]
    y_disp = _run_ffn(offs, x_disp, W1, b1, W2, b2)

    wnb_flat = wnb.reshape(A, 16)
    final = _sc_combine(y_disp, dest3, wnb_flat)

    return (final.reshape(b, s, O),
            lp.reshape(b, s, E),
            w.reshape(b, s, E),
            ti.reshape(b, s, K))
